# Initial kernel scaffold; baseline (speedup 1.0000x reference)
import jax
import jax.numpy as jnp
from jax import lax
from jax.experimental import pallas as pl

N_ANC = 2304
N_PAIR = 8
IB = 128
JB = 768
NMS_TH = 0.5


def _dot_relu_body(a_ref, b_ref, bias_ref, o_ref):
    acc = jnp.dot(a_ref[...], b_ref[...], preferred_element_type=jnp.float32)
    o_ref[...] = jnp.maximum(acc + bias_ref[...], 0.0)


def _dot_relu(a, b, bias, bm):
    m, k = a.shape
    n = b.shape[1]
    return pl.pallas_call(
        _dot_relu_body,
        grid=(m // bm,),
        in_specs=[pl.BlockSpec((bm, k), lambda i: (i, 0)),
                  pl.BlockSpec((k, n), lambda i: (0, 0)),
                  pl.BlockSpec((1, n), lambda i: (0, 0))],
        out_specs=pl.BlockSpec((bm, n), lambda i: (i, 0)),
        out_shape=jax.ShapeDtypeStruct((m, n), jnp.float32),
    )(a, b, bias)


def _patches(x_nhwc, stride):
    b, h, w, c = x_nhwc.shape
    xp = jnp.pad(x_nhwc, ((0, 0), (1, 1), (1, 1), (0, 0)))
    ho, wo = h // stride, w // stride
    cols = []
    for dy in range(3):
        for dx in range(3):
            cols.append(xp[:, dy:dy + stride * ho:stride, dx:dx + stride * wo:stride, :])
    return jnp.concatenate(cols, axis=-1).reshape(b * ho * wo, 9 * c)


def _wmat(w):
    o = w.shape[0]
    return w.transpose(2, 3, 1, 0).reshape(-1, o)


def _heads_body(f_ref, wc_ref, bc_ref, wr_ref, br_ref, g_ref, s_ref, bx_ref):
    f = f_ref[...]
    logits = jnp.dot(f, wc_ref[...], preferred_element_type=jnp.float32) + bc_ref[...]
    m = jnp.max(logits, axis=1, keepdims=True)
    e = jnp.exp(logits - m)
    s = (((e[:, 0:1] + e[:, 1:2]) + e[:, 2:3]) + e[:, 3:4]) + e[:, 4:5]
    s_ref[...] = e / s
    reg = jnp.dot(f, wr_ref[...], preferred_element_type=jnp.float32) + br_ref[...]
    bx_ref[...] = g_ref[...] + reg


def _heads(f3, wc, bc, wr, br, grid_boxes, bm=576):
    m = f3.shape[0]
    return pl.pallas_call(
        _heads_body,
        grid=(m // bm,),
        in_specs=[pl.BlockSpec((bm, 64), lambda i: (i, 0)),
                  pl.BlockSpec((64, 5), lambda i: (0, 0)),
                  pl.BlockSpec((1, 5), lambda i: (0, 0)),
                  pl.BlockSpec((64, 4), lambda i: (0, 0)),
                  pl.BlockSpec((1, 4), lambda i: (0, 0)),
                  pl.BlockSpec((bm, 4), lambda i: (i, 0))],
        out_specs=[pl.BlockSpec((bm, 5), lambda i: (i, 0)),
                   pl.BlockSpec((bm, 4), lambda i: (i, 0))],
        out_shape=[jax.ShapeDtypeStruct((m, 5), jnp.float32),
                   jax.ShapeDtypeStruct((m, 4), jnp.float32)],
    )(f3, wc, bc, wr, br, grid_boxes)


def _rank_body(s_ref, sall_ref, r_ref):
    ib = pl.program_id(0)
    si = s_ref[...][:, :, None]
    sj = sall_ref[...][:, None, :]
    i_idx = ib * IB + lax.broadcasted_iota(jnp.float32, (1, IB, 1), 1)
    j_idx = lax.broadcasted_iota(jnp.float32, (1, 1, N_ANC), 2)
    gt = (sj > si).astype(jnp.float32)
    tie = jnp.logical_and(sj == si, j_idx < i_idx).astype(jnp.float32)
    r_ref[...] = jnp.sum(gt + tie, axis=2)


def _ranks(scores_p):
    return pl.pallas_call(
        _rank_body,
        grid=(N_ANC // IB,),
        in_specs=[pl.BlockSpec((N_PAIR, IB), lambda i: (0, i)),
                  pl.BlockSpec((N_PAIR, N_ANC), lambda i: (0, 0))],
        out_specs=pl.BlockSpec((N_PAIR, IB), lambda i: (0, i)),
        out_shape=jax.ShapeDtypeStruct((N_PAIR, N_ANC), jnp.float32),
    )(scores_p)


def _nms_body(bx_ref, rank_ref, keep_ref, keep2_ref):
    keep_ref[...] = jnp.ones((N_PAIR, N_ANC), jnp.float32)

    def one_sweep():
        for ib in range(N_ANC // IB):
            isl = slice(ib * IB, (ib + 1) * IB)
            x0i = bx_ref[0, :, isl][:, :, None]
            y0i = bx_ref[1, :, isl][:, :, None]
            x1i = bx_ref[2, :, isl][:, :, None]
            y1i = bx_ref[3, :, isl][:, :, None]
            ri = rank_ref[:, isl][:, :, None]
            ai = (x1i - x0i) * (y1i - y0i)
            supp = jnp.zeros((N_PAIR, IB), jnp.float32)
            for jb in range(N_ANC // JB):
                jsl = slice(jb * JB, (jb + 1) * JB)
                x0j = bx_ref[0, :, jsl][:, None, :]
                y0j = bx_ref[1, :, jsl][:, None, :]
                x1j = bx_ref[2, :, jsl][:, None, :]
                y1j = bx_ref[3, :, jsl][:, None, :]
                rj = rank_ref[:, jsl][:, None, :]
                aj = (x1j - x0j) * (y1j - y0j)
                kj = keep_ref[:, jsl][:, None, :]
                iw = jnp.maximum(jnp.minimum(x1i, x1j) - jnp.maximum(x0i, x0j), 0.0)
                ih = jnp.maximum(jnp.minimum(y1i, y1j) - jnp.maximum(y0i, y0j), 0.0)
                inter = iw * ih
                iou = inter / (ai + aj - inter + 1e-9)
                cond = jnp.logical_and(iou > NMS_TH, rj < ri)
                cond = jnp.logical_and(cond, kj > 0.5)
                supp = jnp.maximum(supp, jnp.max(cond.astype(jnp.float32), axis=2))
            keep2_ref[:, isl] = 1.0 - supp

    def cond_fn(carry):
        it, changed = carry
        return jnp.logical_and(changed, it < N_ANC)

    def body_fn(carry):
        it, _ = carry
        one_sweep()
        diff = jnp.max(jnp.abs(keep2_ref[...] - keep_ref[...]))
        keep_ref[...] = keep2_ref[...]
        return it + 1, diff > 0.0

    lax.while_loop(cond_fn, body_fn, (jnp.int32(0), jnp.bool_(True)))


def _nms_keep(box_planes, ranks):
    keep, _ = pl.pallas_call(
        _nms_body,
        in_specs=[pl.BlockSpec((4, N_PAIR, N_ANC), lambda: (0, 0, 0)),
                  pl.BlockSpec((N_PAIR, N_ANC), lambda: (0, 0))],
        out_specs=[pl.BlockSpec((N_PAIR, N_ANC), lambda: (0, 0)),
                   pl.BlockSpec((N_PAIR, N_ANC), lambda: (0, 0))],
        out_shape=[jax.ShapeDtypeStruct((N_PAIR, N_ANC), jnp.float32),
                   jax.ShapeDtypeStruct((N_PAIR, N_ANC), jnp.float32)],
    )(box_planes, ranks)
    return keep


def _gather_body(planes_ref, rank_ref, o_ref):
    rb = pl.program_id(1)
    r_iota = rb * IB + lax.broadcasted_iota(jnp.float32, (IB, 1), 0)
    onehot = rank_ref[0, :][None, :] == r_iota
    keep_s = jnp.sum(jnp.where(onehot, planes_ref[5, 0, :][None, :], 0.0),
                     axis=1, keepdims=True)
    for q in range(5):
        v = jnp.sum(jnp.where(onehot, planes_ref[q, 0, :][None, :], 0.0),
                    axis=1, keepdims=True)
        o_ref[0, :, q:q + 1] = v * keep_s
    o_ref[0, :, 5:6] = keep_s


def _gather(planes, ranks):
    return pl.pallas_call(
        _gather_body,
        grid=(N_PAIR, N_ANC // IB),
        in_specs=[pl.BlockSpec((6, 1, N_ANC), lambda p, r: (0, p, 0)),
                  pl.BlockSpec((1, N_ANC), lambda p, r: (p, 0))],
        out_specs=pl.BlockSpec((1, IB, 6), lambda p, r: (p, r, 0)),
        out_shape=jax.ShapeDtypeStruct((N_PAIR, N_ANC, 6), jnp.float32),
    )(planes, ranks)


def kernel(images, W1, b1, W2, b2, W3, b3, Wc, bc, Wr, br):
    x = images.transpose(0, 2, 3, 1)
    p1 = _patches(x, 2)
    f1 = _dot_relu(p1, _wmat(W1), b1[None, :], bm=1024)
    p2 = _patches(f1.reshape(2, 192, 192, 16), 2)
    f2 = _dot_relu(p2, _wmat(W2), b2[None, :], bm=1024)
    p3 = _patches(f2.reshape(2, 96, 96, 32), 2)
    f3 = _dot_relu(p3, _wmat(W3), b3[None, :], bm=512)

    gy, gx = jnp.meshgrid(jnp.arange(48), jnp.arange(48), indexing='ij')
    grid1 = jnp.stack([gx, gy, gx + 1, gy + 1], axis=-1).reshape(-1, 4).astype(jnp.float32)
    grid_b = jnp.tile(grid1, (2, 1))

    scores, boxes = _heads(f3, Wc.reshape(5, 64).T, bc[None, :],
                           Wr.reshape(4, 64).T, br[None, :], grid_b)

    s_img = scores.reshape(2, N_ANC, 5)
    scores_p = jnp.concatenate([s_img[0, :, 1:].T, s_img[1, :, 1:].T], axis=0)
    bx_img = boxes.reshape(2, N_ANC, 4)
    box_planes = jnp.stack([
        jnp.repeat(bx_img[:, :, q], 4, axis=0) for q in range(4)], axis=0)

    ranks = _ranks(scores_p)
    keep = _nms_keep(box_planes, ranks)

    planes = jnp.concatenate([box_planes,
                              scores_p[None, :, :],
                              keep[None, :, :]], axis=0)
    sorted_pl = _gather(planes, ranks)

    out_boxes = sorted_pl[:, :, 0:4].reshape(2, 4, N_ANC, 4)
    out_scores = sorted_pl[:, :, 4].reshape(2, 4, N_ANC)
    keep_mask = sorted_pl[:, :, 5].reshape(2, 4, N_ANC) > 0.5
    return out_boxes, out_scores, keep_mask


# speedup vs baseline: 17.1507x; 17.1507x over previous
import jax
import jax.numpy as jnp
from jax import lax
from jax.experimental import pallas as pl

N_ANC = 2304
N_PAIR = 8
IB = 128
JB = 768
NMS_TH = 0.5


def _dot_relu_body(a_ref, b_ref, bias_ref, o_ref):
    acc = jnp.dot(a_ref[...], b_ref[...], preferred_element_type=jnp.float32)
    o_ref[...] = jnp.maximum(acc + bias_ref[...], 0.0)


def _dot_relu(a, b, bias, bm):
    m, k = a.shape
    n = b.shape[1]
    return pl.pallas_call(
        _dot_relu_body,
        grid=(m // bm,),
        in_specs=[pl.BlockSpec((bm, k), lambda i: (i, 0)),
                  pl.BlockSpec((k, n), lambda i: (0, 0)),
                  pl.BlockSpec((1, n), lambda i: (0, 0))],
        out_specs=pl.BlockSpec((bm, n), lambda i: (i, 0)),
        out_shape=jax.ShapeDtypeStruct((m, n), jnp.float32),
    )(a, b, bias)


def _patches(x_nhwc, stride):
    b, h, w, c = x_nhwc.shape
    xp = jnp.pad(x_nhwc, ((0, 0), (1, 1), (1, 1), (0, 0)))
    ho, wo = h // stride, w // stride
    cols = []
    for dy in range(3):
        for dx in range(3):
            cols.append(xp[:, dy:dy + stride * ho:stride, dx:dx + stride * wo:stride, :])
    return jnp.concatenate(cols, axis=-1).reshape(b * ho * wo, 9 * c)


def _wmat(w):
    o = w.shape[0]
    return w.transpose(2, 3, 1, 0).reshape(-1, o)


def _heads_body(f_ref, wc_ref, bc_ref, wr_ref, br_ref, g_ref, s_ref, bx_ref):
    f = f_ref[...]
    logits = jnp.dot(f, wc_ref[...], preferred_element_type=jnp.float32) + bc_ref[...]
    m = jnp.max(logits, axis=1, keepdims=True)
    e = jnp.exp(logits - m)
    s = (((e[:, 0:1] + e[:, 1:2]) + e[:, 2:3]) + e[:, 3:4]) + e[:, 4:5]
    s_ref[...] = e / s
    reg = jnp.dot(f, wr_ref[...], preferred_element_type=jnp.float32) + br_ref[...]
    bx_ref[...] = g_ref[...] + reg


def _heads(f3, wc, bc, wr, br, grid_boxes, bm=576):
    m = f3.shape[0]
    return pl.pallas_call(
        _heads_body,
        grid=(m // bm,),
        in_specs=[pl.BlockSpec((bm, 64), lambda i: (i, 0)),
                  pl.BlockSpec((64, 5), lambda i: (0, 0)),
                  pl.BlockSpec((1, 5), lambda i: (0, 0)),
                  pl.BlockSpec((64, 4), lambda i: (0, 0)),
                  pl.BlockSpec((1, 4), lambda i: (0, 0)),
                  pl.BlockSpec((bm, 4), lambda i: (i, 0))],
        out_specs=[pl.BlockSpec((bm, 5), lambda i: (i, 0)),
                   pl.BlockSpec((bm, 4), lambda i: (i, 0))],
        out_shape=[jax.ShapeDtypeStruct((m, 5), jnp.float32),
                   jax.ShapeDtypeStruct((m, 4), jnp.float32)],
    )(f3, wc, bc, wr, br, grid_boxes)


def _rank_body(s_ref, sall_ref, r_ref):
    ib = pl.program_id(0)
    si = s_ref[...][:, :, None]
    sj = sall_ref[...][:, None, :]
    i_idx = ib * IB + lax.broadcasted_iota(jnp.int32, (1, IB, 1), 1)
    j_idx = lax.broadcasted_iota(jnp.int32, (1, 1, N_ANC), 2)
    gt = (sj > si).astype(jnp.float32)
    tie = jnp.logical_and(sj == si, j_idx < i_idx).astype(jnp.float32)
    r_ref[...] = jnp.sum(gt + tie, axis=2)


def _ranks(scores_p):
    return pl.pallas_call(
        _rank_body,
        grid=(N_ANC // IB,),
        in_specs=[pl.BlockSpec((N_PAIR, IB), lambda i: (0, i)),
                  pl.BlockSpec((N_PAIR, N_ANC), lambda i: (0, 0))],
        out_specs=pl.BlockSpec((N_PAIR, IB), lambda i: (0, i)),
        out_shape=jax.ShapeDtypeStruct((N_PAIR, N_ANC), jnp.float32),
    )(scores_p, scores_p)


def _nms_body(bx_ref, rank_ref, keep_ref, keep2_ref):
    keep_ref[...] = jnp.ones((N_PAIR, N_ANC), jnp.float32)

    def one_sweep():
        for ib in range(N_ANC // IB):
            isl = slice(ib * IB, (ib + 1) * IB)
            x0i = bx_ref[0, :, isl][:, :, None]
            y0i = bx_ref[1, :, isl][:, :, None]
            x1i = bx_ref[2, :, isl][:, :, None]
            y1i = bx_ref[3, :, isl][:, :, None]
            ri = rank_ref[:, isl][:, :, None]
            ai = (x1i - x0i) * (y1i - y0i)
            supp = jnp.zeros((N_PAIR, IB), jnp.float32)
            for jb in range(N_ANC // JB):
                jsl = slice(jb * JB, (jb + 1) * JB)
                x0j = bx_ref[0, :, jsl][:, None, :]
                y0j = bx_ref[1, :, jsl][:, None, :]
                x1j = bx_ref[2, :, jsl][:, None, :]
                y1j = bx_ref[3, :, jsl][:, None, :]
                rj = rank_ref[:, jsl][:, None, :]
                aj = (x1j - x0j) * (y1j - y0j)
                kj = keep_ref[:, jsl][:, None, :]
                iw = jnp.maximum(jnp.minimum(x1i, x1j) - jnp.maximum(x0i, x0j), 0.0)
                ih = jnp.maximum(jnp.minimum(y1i, y1j) - jnp.maximum(y0i, y0j), 0.0)
                inter = iw * ih
                iou = inter / (ai + aj - inter + 1e-9)
                cond = jnp.logical_and(iou > NMS_TH, rj < ri)
                cond = jnp.logical_and(cond, kj > 0.5)
                supp = jnp.maximum(supp, jnp.max(cond.astype(jnp.float32), axis=2))
            keep2_ref[:, isl] = 1.0 - supp

    def cond_fn(carry):
        it, changed = carry
        return jnp.logical_and(changed, it < N_ANC)

    def body_fn(carry):
        it, _ = carry
        one_sweep()
        diff = jnp.max(jnp.abs(keep2_ref[...] - keep_ref[...]))
        keep_ref[...] = keep2_ref[...]
        return it + 1, diff > 0.0

    lax.while_loop(cond_fn, body_fn, (jnp.int32(0), jnp.bool_(True)))


def _nms_keep(box_planes, ranks):
    keep, _ = pl.pallas_call(
        _nms_body,
        in_specs=[pl.BlockSpec((4, N_PAIR, N_ANC), lambda: (0, 0, 0)),
                  pl.BlockSpec((N_PAIR, N_ANC), lambda: (0, 0))],
        out_specs=[pl.BlockSpec((N_PAIR, N_ANC), lambda: (0, 0)),
                   pl.BlockSpec((N_PAIR, N_ANC), lambda: (0, 0))],
        out_shape=[jax.ShapeDtypeStruct((N_PAIR, N_ANC), jnp.float32),
                   jax.ShapeDtypeStruct((N_PAIR, N_ANC), jnp.float32)],
    )(box_planes, ranks)
    return keep


def _gather_body(planes_ref, rank_ref, o_ref):
    rb = pl.program_id(1)
    r_iota = (rb * IB + lax.broadcasted_iota(jnp.int32, (IB, 1), 0)).astype(jnp.float32)
    onehot = rank_ref[0, 0, :][None, :] == r_iota
    keep_s = jnp.sum(jnp.where(onehot, planes_ref[0, 5, :][None, :], 0.0),
                     axis=1, keepdims=True)
    for q in range(5):
        v = jnp.sum(jnp.where(onehot, planes_ref[0, q, :][None, :], 0.0),
                    axis=1, keepdims=True)
        o_ref[0, :, q:q + 1] = v * keep_s
    o_ref[0, :, 5:6] = keep_s


def _gather(planes, ranks):
    return pl.pallas_call(
        _gather_body,
        grid=(N_PAIR, N_ANC // IB),
        in_specs=[pl.BlockSpec((1, 6, N_ANC), lambda p, r: (p, 0, 0)),
                  pl.BlockSpec((1, 1, N_ANC), lambda p, r: (p, 0, 0))],
        out_specs=pl.BlockSpec((1, IB, 6), lambda p, r: (p, r, 0)),
        out_shape=jax.ShapeDtypeStruct((N_PAIR, N_ANC, 6), jnp.float32),
    )(planes, ranks)


def kernel(images, W1, b1, W2, b2, W3, b3, Wc, bc, Wr, br):
    x = images.transpose(0, 2, 3, 1)
    p1 = _patches(x, 2)
    f1 = _dot_relu(p1, _wmat(W1), b1[None, :], bm=1024)
    p2 = _patches(f1.reshape(2, 192, 192, 16), 2)
    f2 = _dot_relu(p2, _wmat(W2), b2[None, :], bm=1024)
    p3 = _patches(f2.reshape(2, 96, 96, 32), 2)
    f3 = _dot_relu(p3, _wmat(W3), b3[None, :], bm=512)

    gy, gx = jnp.meshgrid(jnp.arange(48), jnp.arange(48), indexing='ij')
    grid1 = jnp.stack([gx, gy, gx + 1, gy + 1], axis=-1).reshape(-1, 4).astype(jnp.float32)
    grid_b = jnp.tile(grid1, (2, 1))

    scores, boxes = _heads(f3, Wc.reshape(5, 64).T, bc[None, :],
                           Wr.reshape(4, 64).T, br[None, :], grid_b)

    s_img = scores.reshape(2, N_ANC, 5)
    scores_p = jnp.concatenate([s_img[0, :, 1:].T, s_img[1, :, 1:].T], axis=0)
    bx_img = boxes.reshape(2, N_ANC, 4)
    box_planes = jnp.stack([
        jnp.repeat(bx_img[:, :, q], 4, axis=0) for q in range(4)], axis=0)

    ranks = _ranks(scores_p)
    keep = _nms_keep(box_planes, ranks)

    planes = jnp.concatenate([box_planes,
                              scores_p[None, :, :],
                              keep[None, :, :]], axis=0)
    planes = planes.transpose(1, 0, 2)
    sorted_pl = _gather(planes, ranks.reshape(N_PAIR, 1, N_ANC))

    out_boxes = sorted_pl[:, :, 0:4].reshape(2, 4, N_ANC, 4)
    out_scores = sorted_pl[:, :, 4].reshape(2, 4, N_ANC)
    keep_mask = sorted_pl[:, :, 5].reshape(2, 4, N_ANC) > 0.5
    return out_boxes, out_scores, keep_mask


# speedup vs baseline: 17.3173x; 1.0097x over previous
import jax
import jax.numpy as jnp
from jax import lax
from jax.experimental import pallas as pl

N_ANC = 2304
N_PAIR = 8
IB = 128
JB = 768
NMS_TH = 0.5


def _dot_relu_body(a_ref, b_ref, bias_ref, o_ref):
    acc = jnp.dot(a_ref[...], b_ref[...], preferred_element_type=jnp.float32)
    o_ref[...] = jnp.maximum(acc + bias_ref[...], 0.0)


def _dot_relu(a, b, bias, bm):
    m, k = a.shape
    n = b.shape[1]
    return pl.pallas_call(
        _dot_relu_body,
        grid=(m // bm,),
        in_specs=[pl.BlockSpec((bm, k), lambda i: (i, 0)),
                  pl.BlockSpec((k, n), lambda i: (0, 0)),
                  pl.BlockSpec((1, n), lambda i: (0, 0))],
        out_specs=pl.BlockSpec((bm, n), lambda i: (i, 0)),
        out_shape=jax.ShapeDtypeStruct((m, n), jnp.float32),
    )(a, b, bias)


def _patches(x_nhwc, stride):
    b, h, w, c = x_nhwc.shape
    xp = jnp.pad(x_nhwc, ((0, 0), (1, 1), (1, 1), (0, 0)))
    ho, wo = h // stride, w // stride
    cols = []
    for dy in range(3):
        for dx in range(3):
            cols.append(xp[:, dy:dy + stride * ho:stride, dx:dx + stride * wo:stride, :])
    return jnp.concatenate(cols, axis=-1).reshape(b * ho * wo, 9 * c)


def _wmat(w):
    o = w.shape[0]
    return w.transpose(2, 3, 1, 0).reshape(-1, o)


def _heads_body(f_ref, wc_ref, bc_ref, wr_ref, br_ref, g_ref, s_ref, bx_ref):
    f = f_ref[...]
    logits = jnp.dot(f, wc_ref[...], preferred_element_type=jnp.float32) + bc_ref[...]
    m = jnp.max(logits, axis=1, keepdims=True)
    e = jnp.exp(logits - m)
    s = (((e[:, 0:1] + e[:, 1:2]) + e[:, 2:3]) + e[:, 3:4]) + e[:, 4:5]
    s_ref[...] = e / s
    reg = jnp.dot(f, wr_ref[...], preferred_element_type=jnp.float32) + br_ref[...]
    bx_ref[...] = g_ref[...] + reg


def _heads(f3, wc, bc, wr, br, grid_boxes, bm=576):
    m = f3.shape[0]
    return pl.pallas_call(
        _heads_body,
        grid=(m // bm,),
        in_specs=[pl.BlockSpec((bm, 64), lambda i: (i, 0)),
                  pl.BlockSpec((64, 5), lambda i: (0, 0)),
                  pl.BlockSpec((1, 5), lambda i: (0, 0)),
                  pl.BlockSpec((64, 4), lambda i: (0, 0)),
                  pl.BlockSpec((1, 4), lambda i: (0, 0)),
                  pl.BlockSpec((bm, 4), lambda i: (i, 0))],
        out_specs=[pl.BlockSpec((bm, 5), lambda i: (i, 0)),
                   pl.BlockSpec((bm, 4), lambda i: (i, 0))],
        out_shape=[jax.ShapeDtypeStruct((m, 5), jnp.float32),
                   jax.ShapeDtypeStruct((m, 4), jnp.float32)],
    )(f3, wc, bc, wr, br, grid_boxes)


def _rank_body(s_ref, sall_ref, r_ref):
    ib = pl.program_id(0)
    si = s_ref[...][:, :, None]
    sj = sall_ref[...][:, None, :]
    i_idx = ib * IB + lax.broadcasted_iota(jnp.int32, (1, IB, 1), 1)
    j_idx = lax.broadcasted_iota(jnp.int32, (1, 1, N_ANC), 2)
    gt = (sj > si).astype(jnp.float32)
    tie = jnp.logical_and(sj == si, j_idx < i_idx).astype(jnp.float32)
    r_ref[...] = jnp.sum(gt + tie, axis=2)


def _ranks(scores_p):
    return pl.pallas_call(
        _rank_body,
        grid=(N_ANC // IB,),
        in_specs=[pl.BlockSpec((N_PAIR, IB), lambda i: (0, i)),
                  pl.BlockSpec((N_PAIR, N_ANC), lambda i: (0, 0))],
        out_specs=pl.BlockSpec((N_PAIR, IB), lambda i: (0, i)),
        out_shape=jax.ShapeDtypeStruct((N_PAIR, N_ANC), jnp.float32),
    )(scores_p, scores_p)


BIG = 1e9


SB = 8


def _nms_body(bxl_ref, bxt_ref, rank_ref, rankt_ref, keep_ref, keep2t_ref, keept_ref):
    keept_ref[...] = jnp.ones((N_ANC, N_PAIR), jnp.float32)
    rank_lane = rank_ref[...]

    def sweep_block(ib, t_lane):
        bt = bxt_ref[pl.ds(ib * SB, SB), :]
        rt = rankt_ref[pl.ds(ib * SB, SB), :]
        for img in range(2):
            c0 = img * 4
            x0i = bt[:, c0:c0 + 1]
            y0i = bt[:, c0 + 1:c0 + 2]
            x1i = bt[:, c0 + 2:c0 + 3]
            y1i = bt[:, c0 + 3:c0 + 4]
            ai = (x1i - x0i) * (y1i - y0i)
            x0j = bxl_ref[c0:c0 + 1, :]
            y0j = bxl_ref[c0 + 1:c0 + 2, :]
            x1j = bxl_ref[c0 + 2:c0 + 3, :]
            y1j = bxl_ref[c0 + 3:c0 + 4, :]
            aj = (x1j - x0j) * (y1j - y0j)
            iw = jnp.maximum(jnp.minimum(x1i, x1j) - jnp.maximum(x0i, x0j), 0.0)
            ih = jnp.maximum(jnp.minimum(y1i, y1j) - jnp.maximum(y0i, y0j), 0.0)
            inter = iw * ih
            u = ai + aj - inter + 1e-9
            cond = jnp.logical_and(inter + inter > u, u > 0.0)
            for q in range(4):
                p = img * 4 + q
                c = jnp.where(cond, t_lane[p:p + 1, :], BIG)
                m = jnp.min(c, axis=1, keepdims=True)
                newk = (m >= rt[:, p:p + 1]).astype(jnp.float32)
                keep2t_ref[pl.ds(ib * SB, SB), p:p + 1] = newk
        return t_lane

    def cond_fn(carry):
        it, changed = carry
        return jnp.logical_and(changed, it < N_ANC)

    def body_fn(carry):
        it, _ = carry
        kt = keept_ref[...]
        t_lane = jnp.where(jnp.transpose(kt) > 0.5, rank_lane, BIG)
        lax.fori_loop(0, N_ANC // SB, sweep_block, t_lane)
        diff = jnp.max(jnp.abs(keep2t_ref[...] - keept_ref[...]))
        keept_ref[...] = keep2t_ref[...]
        return it + 1, diff > 0.0

    lax.while_loop(cond_fn, body_fn, (jnp.int32(0), jnp.bool_(True)))
    keep_ref[...] = jnp.transpose(keept_ref[...])


def _nms_keep(boxes_l, boxes_t, ranks, ranks_t):
    from jax.experimental.pallas import tpu as pltpu
    keep, _ = pl.pallas_call(
        _nms_body,
        in_specs=[pl.BlockSpec((N_PAIR, N_ANC), lambda: (0, 0)),
                  pl.BlockSpec((N_ANC, 8), lambda: (0, 0)),
                  pl.BlockSpec((N_PAIR, N_ANC), lambda: (0, 0)),
                  pl.BlockSpec((N_ANC, N_PAIR), lambda: (0, 0))],
        out_specs=[pl.BlockSpec((N_PAIR, N_ANC), lambda: (0, 0)),
                   pl.BlockSpec((N_ANC, N_PAIR), lambda: (0, 0))],
        out_shape=[jax.ShapeDtypeStruct((N_PAIR, N_ANC), jnp.float32),
                   jax.ShapeDtypeStruct((N_ANC, N_PAIR), jnp.float32)],
        scratch_shapes=[pltpu.VMEM((N_ANC, N_PAIR), jnp.float32)],
    )(boxes_l, boxes_t, ranks, ranks_t)
    return keep


def _gather_body(planes_ref, rank_ref, o_ref):
    rb = pl.program_id(1)
    r_iota = (rb * IB + lax.broadcasted_iota(jnp.int32, (IB, 1), 0)).astype(jnp.float32)
    onehot = (rank_ref[0, 0, :][None, :] == r_iota).astype(jnp.bfloat16)
    x = planes_ref[0]
    hi = x.astype(jnp.bfloat16)
    r1 = x - hi.astype(jnp.float32)
    mid = r1.astype(jnp.bfloat16)
    lo = (r1 - mid.astype(jnp.float32)).astype(jnp.bfloat16)
    dn = (((1,), (1,)), ((), ()))
    gh = lax.dot_general(onehot, hi, dn, preferred_element_type=jnp.float32)
    gm = lax.dot_general(onehot, mid, dn, preferred_element_type=jnp.float32)
    gl = lax.dot_general(onehot, lo, dn, preferred_element_type=jnp.float32)
    g = gh + (gm + gl)
    keep_s = g[:, 5:6]
    o_ref[0, :, 0:5] = g[:, 0:5] * keep_s
    o_ref[0, :, 5:6] = keep_s


def _gather(planes, ranks):
    return pl.pallas_call(
        _gather_body,
        grid=(N_PAIR, N_ANC // IB),
        in_specs=[pl.BlockSpec((1, 6, N_ANC), lambda p, r: (p, 0, 0)),
                  pl.BlockSpec((1, 1, N_ANC), lambda p, r: (p, 0, 0))],
        out_specs=pl.BlockSpec((1, IB, 6), lambda p, r: (p, r, 0)),
        out_shape=jax.ShapeDtypeStruct((N_PAIR, N_ANC, 6), jnp.float32),
    )(planes, ranks)


def kernel(images, W1, b1, W2, b2, W3, b3, Wc, bc, Wr, br):
    x = images.transpose(0, 2, 3, 1)
    p1 = _patches(x, 2)
    f1 = _dot_relu(p1, _wmat(W1), b1[None, :], bm=1024)
    p2 = _patches(f1.reshape(2, 192, 192, 16), 2)
    f2 = _dot_relu(p2, _wmat(W2), b2[None, :], bm=1024)
    p3 = _patches(f2.reshape(2, 96, 96, 32), 2)
    f3 = _dot_relu(p3, _wmat(W3), b3[None, :], bm=512)

    gy, gx = jnp.meshgrid(jnp.arange(48), jnp.arange(48), indexing='ij')
    grid1 = jnp.stack([gx, gy, gx + 1, gy + 1], axis=-1).reshape(-1, 4).astype(jnp.float32)
    grid_b = jnp.tile(grid1, (2, 1))

    scores, boxes = _heads(f3, Wc.reshape(5, 64).T, bc[None, :],
                           Wr.reshape(4, 64).T, br[None, :], grid_b)

    s_img = scores.reshape(2, N_ANC, 5)
    scores_p = jnp.concatenate([s_img[0, :, 1:].T, s_img[1, :, 1:].T], axis=0)
    bx_img = boxes.reshape(2, N_ANC, 4)
    box_planes = jnp.stack([
        jnp.repeat(bx_img[:, :, q], 4, axis=0) for q in range(4)], axis=0)

    ranks = _ranks(scores_p)
    keep = _nms_keep(bx_img.transpose(0, 2, 1).reshape(8, N_ANC),
                     bx_img.transpose(1, 0, 2).reshape(N_ANC, 8),
                     ranks, ranks.T)

    planes = jnp.concatenate([box_planes,
                              scores_p[None, :, :],
                              keep[None, :, :]], axis=0)
    planes = planes.transpose(1, 0, 2)
    sorted_pl = _gather(planes, ranks.reshape(N_PAIR, 1, N_ANC))

    out_boxes = sorted_pl[:, :, 0:4].reshape(2, 4, N_ANC, 4)
    out_scores = sorted_pl[:, :, 4].reshape(2, 4, N_ANC)
    keep_mask = sorted_pl[:, :, 5].reshape(2, 4, N_ANC) > 0.5
    return out_boxes, out_scores, keep_mask


# speedup vs baseline: 18.6066x; 1.0745x over previous
import jax
import jax.numpy as jnp
from jax import lax
from jax.experimental import pallas as pl

N_ANC = 2304
N_PAIR = 8
IB = 128
JB = 768
NMS_TH = 0.5


def _dot_relu_body(a_ref, b_ref, bias_ref, o_ref):
    acc = jnp.dot(a_ref[...], b_ref[...], preferred_element_type=jnp.float32)
    o_ref[...] = jnp.maximum(acc + bias_ref[...], 0.0)


def _dot_relu(a, b, bias, bm):
    m, k = a.shape
    n = b.shape[1]
    return pl.pallas_call(
        _dot_relu_body,
        grid=(m // bm,),
        in_specs=[pl.BlockSpec((bm, k), lambda i: (i, 0)),
                  pl.BlockSpec((k, n), lambda i: (0, 0)),
                  pl.BlockSpec((1, n), lambda i: (0, 0))],
        out_specs=pl.BlockSpec((bm, n), lambda i: (i, 0)),
        out_shape=jax.ShapeDtypeStruct((m, n), jnp.float32),
    )(a, b, bias)


def _tdot_relu_body(a_ref, b_ref, bias_ref, o_ref):
    acc = lax.dot_general(a_ref[...], b_ref[...], (((0,), (0,)), ((), ())),
                          preferred_element_type=jnp.float32)
    o_ref[...] = jnp.maximum(acc + bias_ref[...], 0.0)


def _tdot_relu(a_km, b_kn, bias, bm):
    k, m = a_km.shape
    n = b_kn.shape[1]
    return pl.pallas_call(
        _tdot_relu_body,
        grid=(m // bm,),
        in_specs=[pl.BlockSpec((k, bm), lambda i: (0, i)),
                  pl.BlockSpec((k, n), lambda i: (0, 0)),
                  pl.BlockSpec((1, n), lambda i: (0, 0))],
        out_specs=pl.BlockSpec((bm, n), lambda i: (i, 0)),
        out_shape=jax.ShapeDtypeStruct((m, n), jnp.float32),
    )(a_km, b_kn, bias)


def _patches_nchw(x_nchw):
    b, c, h, w = x_nchw.shape
    xp = jnp.pad(x_nchw, ((0, 0), (0, 0), (1, 1), (1, 1)))
    ho, wo = h // 2, w // 2
    pieces = []
    for dy in range(3):
        for dx in range(3):
            for ci in range(c):
                pieces.append(xp[:, ci, dy:dy + 2 * ho:2, dx:dx + 2 * wo:2])
    return jnp.stack(pieces, axis=0).reshape(9 * c, b * ho * wo)


def _patches_t(x_nhwc, stride):
    b, h, w, c = x_nhwc.shape
    xp = jnp.pad(x_nhwc, ((0, 0), (1, 1), (1, 1), (0, 0)))
    ho, wo = h // stride, w // stride
    cols = []
    for dy in range(3):
        for dx in range(3):
            cols.append(xp[:, dy:dy + stride * ho:stride, dx:dx + stride * wo:stride, :])
    g = jnp.stack(cols, axis=0).reshape(9, b * ho * wo, c)
    return g.transpose(0, 2, 1).reshape(9 * c, b * ho * wo)


def _wmat(w):
    o = w.shape[0]
    return w.transpose(2, 3, 1, 0).reshape(-1, o)


def _heads_body(f_ref, wc_ref, bc_ref, wr_ref, br_ref, g_ref, s_ref, bx_ref):
    f = f_ref[...]
    logits = jnp.dot(f, wc_ref[...], preferred_element_type=jnp.float32) + bc_ref[...]
    m = jnp.max(logits, axis=1, keepdims=True)
    e = jnp.exp(logits - m)
    s = (((e[:, 0:1] + e[:, 1:2]) + e[:, 2:3]) + e[:, 3:4]) + e[:, 4:5]
    s_ref[...] = e / s
    reg = jnp.dot(f, wr_ref[...], preferred_element_type=jnp.float32) + br_ref[...]
    bx_ref[...] = g_ref[...] + reg


def _heads(f3, wc, bc, wr, br, grid_boxes, bm=576):
    m = f3.shape[0]
    return pl.pallas_call(
        _heads_body,
        grid=(m // bm,),
        in_specs=[pl.BlockSpec((bm, 64), lambda i: (i, 0)),
                  pl.BlockSpec((64, 5), lambda i: (0, 0)),
                  pl.BlockSpec((1, 5), lambda i: (0, 0)),
                  pl.BlockSpec((64, 4), lambda i: (0, 0)),
                  pl.BlockSpec((1, 4), lambda i: (0, 0)),
                  pl.BlockSpec((bm, 4), lambda i: (i, 0))],
        out_specs=[pl.BlockSpec((bm, 5), lambda i: (i, 0)),
                   pl.BlockSpec((bm, 4), lambda i: (i, 0))],
        out_shape=[jax.ShapeDtypeStruct((m, 5), jnp.float32),
                   jax.ShapeDtypeStruct((m, 4), jnp.float32)],
    )(f3, wc, bc, wr, br, grid_boxes)


def _rank_body(s_ref, sall_ref, r_ref):
    ib = pl.program_id(0)
    si = s_ref[...][:, :, None]
    sj = sall_ref[...][:, None, :]
    i_idx = ib * IB + lax.broadcasted_iota(jnp.int32, (1, IB, 1), 1)
    j_idx = lax.broadcasted_iota(jnp.int32, (1, 1, N_ANC), 2)
    gt = (sj > si).astype(jnp.float32)
    tie = jnp.logical_and(sj == si, j_idx < i_idx).astype(jnp.float32)
    r_ref[...] = jnp.sum(gt + tie, axis=2)


def _ranks(scores_p):
    return pl.pallas_call(
        _rank_body,
        grid=(N_ANC // IB,),
        in_specs=[pl.BlockSpec((N_PAIR, IB), lambda i: (0, i)),
                  pl.BlockSpec((N_PAIR, N_ANC), lambda i: (0, 0))],
        out_specs=pl.BlockSpec((N_PAIR, IB), lambda i: (0, i)),
        out_shape=jax.ShapeDtypeStruct((N_PAIR, N_ANC), jnp.float32),
    )(scores_p, scores_p)


BIG = 1e9


SB = 8


def _nms_body(bxl_ref, bxt_ref, rank_ref, rankt_ref, keep_ref, keep2t_ref, keept_ref):
    keept_ref[...] = jnp.ones((N_ANC, N_PAIR), jnp.float32)
    rank_lane = rank_ref[...]

    def sweep_block(ib, t_lane):
        bt = bxt_ref[pl.ds(ib * SB, SB), :]
        rt = rankt_ref[pl.ds(ib * SB, SB), :]
        for img in range(2):
            c0 = img * 4
            x0i = bt[:, c0:c0 + 1]
            y0i = bt[:, c0 + 1:c0 + 2]
            x1i = bt[:, c0 + 2:c0 + 3]
            y1i = bt[:, c0 + 3:c0 + 4]
            ai = (x1i - x0i) * (y1i - y0i)
            x0j = bxl_ref[c0:c0 + 1, :]
            y0j = bxl_ref[c0 + 1:c0 + 2, :]
            x1j = bxl_ref[c0 + 2:c0 + 3, :]
            y1j = bxl_ref[c0 + 3:c0 + 4, :]
            aj = (x1j - x0j) * (y1j - y0j)
            iw = jnp.maximum(jnp.minimum(x1i, x1j) - jnp.maximum(x0i, x0j), 0.0)
            ih = jnp.maximum(jnp.minimum(y1i, y1j) - jnp.maximum(y0i, y0j), 0.0)
            inter = iw * ih
            u = ai + aj - inter + 1e-9
            cond = jnp.logical_and(inter + inter > u, u > 0.0)
            for q in range(4):
                p = img * 4 + q
                c = jnp.where(cond, t_lane[p:p + 1, :], BIG)
                m = jnp.min(c, axis=1, keepdims=True)
                newk = (m >= rt[:, p:p + 1]).astype(jnp.float32)
                keep2t_ref[pl.ds(ib * SB, SB), p:p + 1] = newk
        return t_lane

    def cond_fn(carry):
        it, changed = carry
        return jnp.logical_and(changed, it < N_ANC)

    def body_fn(carry):
        it, _ = carry
        kt = keept_ref[...]
        t_lane = jnp.where(jnp.transpose(kt) > 0.5, rank_lane, BIG)
        lax.fori_loop(0, N_ANC // SB, sweep_block, t_lane)
        diff = jnp.max(jnp.abs(keep2t_ref[...] - keept_ref[...]))
        keept_ref[...] = keep2t_ref[...]
        return it + 1, diff > 0.0

    lax.while_loop(cond_fn, body_fn, (jnp.int32(0), jnp.bool_(True)))
    keep_ref[...] = jnp.transpose(keept_ref[...])


def _nms_keep(boxes_l, boxes_t, ranks, ranks_t):
    from jax.experimental.pallas import tpu as pltpu
    keep, _ = pl.pallas_call(
        _nms_body,
        in_specs=[pl.BlockSpec((N_PAIR, N_ANC), lambda: (0, 0)),
                  pl.BlockSpec((N_ANC, 8), lambda: (0, 0)),
                  pl.BlockSpec((N_PAIR, N_ANC), lambda: (0, 0)),
                  pl.BlockSpec((N_ANC, N_PAIR), lambda: (0, 0))],
        out_specs=[pl.BlockSpec((N_PAIR, N_ANC), lambda: (0, 0)),
                   pl.BlockSpec((N_ANC, N_PAIR), lambda: (0, 0))],
        out_shape=[jax.ShapeDtypeStruct((N_PAIR, N_ANC), jnp.float32),
                   jax.ShapeDtypeStruct((N_ANC, N_PAIR), jnp.float32)],
        scratch_shapes=[pltpu.VMEM((N_ANC, N_PAIR), jnp.float32)],
    )(boxes_l, boxes_t, ranks, ranks_t)
    return keep


def _gather_body(planes_ref, rank_ref, o_ref):
    rb = pl.program_id(1)
    r_iota = (rb * IB + lax.broadcasted_iota(jnp.int32, (IB, 1), 0)).astype(jnp.float32)
    onehot = (rank_ref[0, 0, :][None, :] == r_iota).astype(jnp.bfloat16)
    x = planes_ref[0]
    hi = x.astype(jnp.bfloat16)
    r1 = x - hi.astype(jnp.float32)
    mid = r1.astype(jnp.bfloat16)
    lo = (r1 - mid.astype(jnp.float32)).astype(jnp.bfloat16)
    dn = (((1,), (1,)), ((), ()))
    gh = lax.dot_general(onehot, hi, dn, preferred_element_type=jnp.float32)
    gm = lax.dot_general(onehot, mid, dn, preferred_element_type=jnp.float32)
    gl = lax.dot_general(onehot, lo, dn, preferred_element_type=jnp.float32)
    g = gh + (gm + gl)
    keep_s = g[:, 5:6]
    o_ref[0, :, 0:5] = g[:, 0:5] * keep_s
    o_ref[0, :, 5:6] = keep_s


def _gather(planes, ranks):
    return pl.pallas_call(
        _gather_body,
        grid=(N_PAIR, N_ANC // IB),
        in_specs=[pl.BlockSpec((1, 6, N_ANC), lambda p, r: (p, 0, 0)),
                  pl.BlockSpec((1, 1, N_ANC), lambda p, r: (p, 0, 0))],
        out_specs=pl.BlockSpec((1, IB, 6), lambda p, r: (p, r, 0)),
        out_shape=jax.ShapeDtypeStruct((N_PAIR, N_ANC, 6), jnp.float32),
    )(planes, ranks)


def kernel(images, W1, b1, W2, b2, W3, b3, Wc, bc, Wr, br):
    p1 = _patches_nchw(images)
    f1 = _tdot_relu(p1, _wmat(W1), b1[None, :], bm=4096)
    p2 = _patches_t(f1.reshape(2, 192, 192, 16), 2)
    f2 = _tdot_relu(p2, _wmat(W2), b2[None, :], bm=2048)
    p3 = _patches_t(f2.reshape(2, 96, 96, 32), 2)
    f3 = _tdot_relu(p3, _wmat(W3), b3[None, :], bm=512)

    gy, gx = jnp.meshgrid(jnp.arange(48), jnp.arange(48), indexing='ij')
    grid1 = jnp.stack([gx, gy, gx + 1, gy + 1], axis=-1).reshape(-1, 4).astype(jnp.float32)
    grid_b = jnp.tile(grid1, (2, 1))

    scores, boxes = _heads(f3, Wc.reshape(5, 64).T, bc[None, :],
                           Wr.reshape(4, 64).T, br[None, :], grid_b)

    s_img = scores.reshape(2, N_ANC, 5)
    scores_p = jnp.concatenate([s_img[0, :, 1:].T, s_img[1, :, 1:].T], axis=0)
    bx_img = boxes.reshape(2, N_ANC, 4)
    box_planes = jnp.stack([
        jnp.repeat(bx_img[:, :, q], 4, axis=0) for q in range(4)], axis=0)

    ranks = _ranks(scores_p)
    keep = _nms_keep(bx_img.transpose(0, 2, 1).reshape(8, N_ANC),
                     bx_img.transpose(1, 0, 2).reshape(N_ANC, 8),
                     ranks, ranks.T)

    planes = jnp.concatenate([box_planes,
                              scores_p[None, :, :],
                              keep[None, :, :]], axis=0)
    planes = planes.transpose(1, 0, 2)
    sorted_pl = _gather(planes, ranks.reshape(N_PAIR, 1, N_ANC))

    out_boxes = sorted_pl[:, :, 0:4].reshape(2, 4, N_ANC, 4)
    out_scores = sorted_pl[:, :, 4].reshape(2, 4, N_ANC)
    keep_mask = sorted_pl[:, :, 5].reshape(2, 4, N_ANC) > 0.5
    return out_boxes, out_scores, keep_mask


# speedup vs baseline: 35.6799x; 1.9176x over previous
import jax
import jax.numpy as jnp
from jax import lax
from jax.experimental import pallas as pl

N_ANC = 2304
N_PAIR = 8
IB = 128
JB = 768
NMS_TH = 0.5


def _dot_relu_body(a_ref, b_ref, bias_ref, o_ref):
    acc = jnp.dot(a_ref[...], b_ref[...], preferred_element_type=jnp.float32)
    o_ref[...] = jnp.maximum(acc + bias_ref[...], 0.0)


def _dot_relu(a, b, bias, bm):
    m, k = a.shape
    n = b.shape[1]
    return pl.pallas_call(
        _dot_relu_body,
        grid=(m // bm,),
        in_specs=[pl.BlockSpec((bm, k), lambda i: (i, 0)),
                  pl.BlockSpec((k, n), lambda i: (0, 0)),
                  pl.BlockSpec((1, n), lambda i: (0, 0))],
        out_specs=pl.BlockSpec((bm, n), lambda i: (i, 0)),
        out_shape=jax.ShapeDtypeStruct((m, n), jnp.float32),
    )(a, b, bias)


def _tdot_relu_body(a_ref, b_ref, bias_ref, o_ref):
    acc = lax.dot_general(a_ref[...], b_ref[...], (((0,), (0,)), ((), ())),
                          preferred_element_type=jnp.float32)
    o_ref[...] = jnp.maximum(acc + bias_ref[...], 0.0)


def _tdot_relu(a_km, b_kn, bias, bm):
    k, m = a_km.shape
    n = b_kn.shape[1]
    return pl.pallas_call(
        _tdot_relu_body,
        grid=(m // bm,),
        in_specs=[pl.BlockSpec((k, bm), lambda i: (0, i)),
                  pl.BlockSpec((k, n), lambda i: (0, 0)),
                  pl.BlockSpec((1, n), lambda i: (0, 0))],
        out_specs=pl.BlockSpec((bm, n), lambda i: (i, 0)),
        out_shape=jax.ShapeDtypeStruct((m, n), jnp.float32),
    )(a_km, b_kn, bias)


def _tdot_relu_t_body(a_ref, b_ref, bias_ref, o_ref):
    acc = lax.dot_general(b_ref[...], a_ref[...], (((0,), (0,)), ((), ())),
                          preferred_element_type=jnp.float32)
    o_ref[...] = jnp.maximum(acc + bias_ref[...], 0.0)


def _tdot_relu_t(a_km, b_kn, bias_n1, bm):
    k, m = a_km.shape
    n = b_kn.shape[1]
    return pl.pallas_call(
        _tdot_relu_t_body,
        grid=(m // bm,),
        in_specs=[pl.BlockSpec((k, bm), lambda i: (0, i)),
                  pl.BlockSpec((k, n), lambda i: (0, 0)),
                  pl.BlockSpec((n, 1), lambda i: (0, 0))],
        out_specs=pl.BlockSpec((n, bm), lambda i: (0, i)),
        out_shape=jax.ShapeDtypeStruct((n, m), jnp.float32),
    )(a_km, b_kn, bias_n1)


def _patches_cm(x, leading_cb):
    d0, d1, h, w = x.shape
    c = d0 if leading_cb else d1
    ho, wo = h // 2, w // 2
    xp = jnp.pad(x, ((0, 0), (0, 0), (1, 1), (1, 1)))
    ev = xp[:, :, :, 0::2]
    od = xp[:, :, :, 1::2]
    wtap = {0: ev[..., 0:wo], 1: od[..., 0:wo], 2: ev[..., 1:wo + 1]}
    pieces = []
    for dy in range(3):
        for dx in range(3):
            pieces.append(wtap[dx][:, :, dy:dy + 2 * ho:2, :])
    g = jnp.stack(pieces, axis=0)
    if not leading_cb:
        g = g.transpose(0, 2, 1, 3, 4)
    return g.reshape(9 * c, -1)


def _wmat(w):
    o = w.shape[0]
    return w.transpose(2, 3, 1, 0).reshape(-1, o)


def _heads_body(f_ref, wc_ref, bc_ref, wr_ref, br_ref, g_ref, s_ref, bx_ref):
    f = f_ref[...]
    logits = jnp.dot(f, wc_ref[...], preferred_element_type=jnp.float32) + bc_ref[...]
    m = jnp.max(logits, axis=1, keepdims=True)
    e = jnp.exp(logits - m)
    s = (((e[:, 0:1] + e[:, 1:2]) + e[:, 2:3]) + e[:, 3:4]) + e[:, 4:5]
    s_ref[...] = e / s
    reg = jnp.dot(f, wr_ref[...], preferred_element_type=jnp.float32) + br_ref[...]
    bx_ref[...] = g_ref[...] + reg


def _heads(f3, wc, bc, wr, br, grid_boxes, bm=576):
    m = f3.shape[0]
    return pl.pallas_call(
        _heads_body,
        grid=(m // bm,),
        in_specs=[pl.BlockSpec((bm, 64), lambda i: (i, 0)),
                  pl.BlockSpec((64, 5), lambda i: (0, 0)),
                  pl.BlockSpec((1, 5), lambda i: (0, 0)),
                  pl.BlockSpec((64, 4), lambda i: (0, 0)),
                  pl.BlockSpec((1, 4), lambda i: (0, 0)),
                  pl.BlockSpec((bm, 4), lambda i: (i, 0))],
        out_specs=[pl.BlockSpec((bm, 5), lambda i: (i, 0)),
                   pl.BlockSpec((bm, 4), lambda i: (i, 0))],
        out_shape=[jax.ShapeDtypeStruct((m, 5), jnp.float32),
                   jax.ShapeDtypeStruct((m, 4), jnp.float32)],
    )(f3, wc, bc, wr, br, grid_boxes)


def _rank_body(s_ref, sall_ref, r_ref):
    ib = pl.program_id(0)
    si = s_ref[...][:, :, None]
    sj = sall_ref[...][:, None, :]
    i_idx = ib * IB + lax.broadcasted_iota(jnp.int32, (1, IB, 1), 1)
    j_idx = lax.broadcasted_iota(jnp.int32, (1, 1, N_ANC), 2)
    gt = (sj > si).astype(jnp.float32)
    tie = jnp.logical_and(sj == si, j_idx < i_idx).astype(jnp.float32)
    r_ref[...] = jnp.sum(gt + tie, axis=2)


def _ranks(scores_p):
    return pl.pallas_call(
        _rank_body,
        grid=(N_ANC // IB,),
        in_specs=[pl.BlockSpec((N_PAIR, IB), lambda i: (0, i)),
                  pl.BlockSpec((N_PAIR, N_ANC), lambda i: (0, 0))],
        out_specs=pl.BlockSpec((N_PAIR, IB), lambda i: (0, i)),
        out_shape=jax.ShapeDtypeStruct((N_PAIR, N_ANC), jnp.float32),
    )(scores_p, scores_p)


BIG = 1e9


SB = 8


def _nms_body(bxl_ref, bxt_ref, rank_ref, rankt_ref, keep_ref, keep2t_ref, keept_ref):
    keept_ref[...] = jnp.ones((N_ANC, N_PAIR), jnp.float32)
    rank_lane = rank_ref[...]

    def sweep_block(ib, t_lane):
        bt = bxt_ref[pl.ds(ib * SB, SB), :]
        rt = rankt_ref[pl.ds(ib * SB, SB), :]
        for img in range(2):
            c0 = img * 4
            x0i = bt[:, c0:c0 + 1]
            y0i = bt[:, c0 + 1:c0 + 2]
            x1i = bt[:, c0 + 2:c0 + 3]
            y1i = bt[:, c0 + 3:c0 + 4]
            ai = (x1i - x0i) * (y1i - y0i)
            x0j = bxl_ref[c0:c0 + 1, :]
            y0j = bxl_ref[c0 + 1:c0 + 2, :]
            x1j = bxl_ref[c0 + 2:c0 + 3, :]
            y1j = bxl_ref[c0 + 3:c0 + 4, :]
            aj = (x1j - x0j) * (y1j - y0j)
            iw = jnp.maximum(jnp.minimum(x1i, x1j) - jnp.maximum(x0i, x0j), 0.0)
            ih = jnp.maximum(jnp.minimum(y1i, y1j) - jnp.maximum(y0i, y0j), 0.0)
            inter = iw * ih
            u = ai + aj - inter + 1e-9
            cond = jnp.logical_and(inter + inter > u, u > 0.0)
            for q in range(4):
                p = img * 4 + q
                c = jnp.where(cond, t_lane[p:p + 1, :], BIG)
                m = jnp.min(c, axis=1, keepdims=True)
                newk = (m >= rt[:, p:p + 1]).astype(jnp.float32)
                keep2t_ref[pl.ds(ib * SB, SB), p:p + 1] = newk
        return t_lane

    def cond_fn(carry):
        it, changed = carry
        return jnp.logical_and(changed, it < N_ANC)

    def body_fn(carry):
        it, _ = carry
        kt = keept_ref[...]
        t_lane = jnp.where(jnp.transpose(kt) > 0.5, rank_lane, BIG)
        lax.fori_loop(0, N_ANC // SB, sweep_block, t_lane)
        diff = jnp.max(jnp.abs(keep2t_ref[...] - keept_ref[...]))
        keept_ref[...] = keep2t_ref[...]
        return it + 1, diff > 0.0

    lax.while_loop(cond_fn, body_fn, (jnp.int32(0), jnp.bool_(True)))
    keep_ref[...] = jnp.transpose(keept_ref[...])


def _nms_keep(boxes_l, boxes_t, ranks, ranks_t):
    from jax.experimental.pallas import tpu as pltpu
    keep, _ = pl.pallas_call(
        _nms_body,
        in_specs=[pl.BlockSpec((N_PAIR, N_ANC), lambda: (0, 0)),
                  pl.BlockSpec((N_ANC, 8), lambda: (0, 0)),
                  pl.BlockSpec((N_PAIR, N_ANC), lambda: (0, 0)),
                  pl.BlockSpec((N_ANC, N_PAIR), lambda: (0, 0))],
        out_specs=[pl.BlockSpec((N_PAIR, N_ANC), lambda: (0, 0)),
                   pl.BlockSpec((N_ANC, N_PAIR), lambda: (0, 0))],
        out_shape=[jax.ShapeDtypeStruct((N_PAIR, N_ANC), jnp.float32),
                   jax.ShapeDtypeStruct((N_ANC, N_PAIR), jnp.float32)],
        scratch_shapes=[pltpu.VMEM((N_ANC, N_PAIR), jnp.float32)],
    )(boxes_l, boxes_t, ranks, ranks_t)
    return keep


def _gather_body(planes_ref, rank_ref, o_ref):
    rb = pl.program_id(1)
    r_iota = (rb * IB + lax.broadcasted_iota(jnp.int32, (IB, 1), 0)).astype(jnp.float32)
    onehot = (rank_ref[0, 0, :][None, :] == r_iota).astype(jnp.bfloat16)
    x = planes_ref[0]
    hi = x.astype(jnp.bfloat16)
    r1 = x - hi.astype(jnp.float32)
    mid = r1.astype(jnp.bfloat16)
    lo = (r1 - mid.astype(jnp.float32)).astype(jnp.bfloat16)
    dn = (((1,), (1,)), ((), ()))
    gh = lax.dot_general(onehot, hi, dn, preferred_element_type=jnp.float32)
    gm = lax.dot_general(onehot, mid, dn, preferred_element_type=jnp.float32)
    gl = lax.dot_general(onehot, lo, dn, preferred_element_type=jnp.float32)
    g = gh + (gm + gl)
    keep_s = g[:, 5:6]
    o_ref[0, :, 0:5] = g[:, 0:5] * keep_s
    o_ref[0, :, 5:6] = keep_s


def _gather(planes, ranks):
    return pl.pallas_call(
        _gather_body,
        grid=(N_PAIR, N_ANC // IB),
        in_specs=[pl.BlockSpec((1, 6, N_ANC), lambda p, r: (p, 0, 0)),
                  pl.BlockSpec((1, 1, N_ANC), lambda p, r: (p, 0, 0))],
        out_specs=pl.BlockSpec((1, IB, 6), lambda p, r: (p, r, 0)),
        out_shape=jax.ShapeDtypeStruct((N_PAIR, N_ANC, 6), jnp.float32),
    )(planes, ranks)


def kernel(images, W1, b1, W2, b2, W3, b3, Wc, bc, Wr, br):
    p1 = _patches_cm(images, leading_cb=False)
    f1t = _tdot_relu_t(p1, _wmat(W1), b1[:, None], bm=4096)
    p2 = _patches_cm(f1t.reshape(16, 2, 192, 192), True)
    f2t = _tdot_relu_t(p2, _wmat(W2), b2[:, None], bm=2048)
    p3 = _patches_cm(f2t.reshape(32, 2, 96, 96), True)
    f3 = _tdot_relu(p3, _wmat(W3), b3[None, :], bm=512)

    gy, gx = jnp.meshgrid(jnp.arange(48), jnp.arange(48), indexing='ij')
    grid1 = jnp.stack([gx, gy, gx + 1, gy + 1], axis=-1).reshape(-1, 4).astype(jnp.float32)
    grid_b = jnp.tile(grid1, (2, 1))

    scores, boxes = _heads(f3, Wc.reshape(5, 64).T, bc[None, :],
                           Wr.reshape(4, 64).T, br[None, :], grid_b)

    s_img = scores.reshape(2, N_ANC, 5)
    scores_p = jnp.concatenate([s_img[0, :, 1:].T, s_img[1, :, 1:].T], axis=0)
    bx_img = boxes.reshape(2, N_ANC, 4)
    box_planes = jnp.stack([
        jnp.repeat(bx_img[:, :, q], 4, axis=0) for q in range(4)], axis=0)

    ranks = _ranks(scores_p)
    keep = _nms_keep(bx_img.transpose(0, 2, 1).reshape(8, N_ANC),
                     bx_img.transpose(1, 0, 2).reshape(N_ANC, 8),
                     ranks, ranks.T)

    planes = jnp.concatenate([box_planes,
                              scores_p[None, :, :],
                              keep[None, :, :]], axis=0)
    planes = planes.transpose(1, 0, 2)
    sorted_pl = _gather(planes, ranks.reshape(N_PAIR, 1, N_ANC))

    out_boxes = sorted_pl[:, :, 0:4].reshape(2, 4, N_ANC, 4)
    out_scores = sorted_pl[:, :, 4].reshape(2, 4, N_ANC)
    keep_mask = sorted_pl[:, :, 5].reshape(2, 4, N_ANC) > 0.5
    return out_boxes, out_scores, keep_mask


# speedup vs baseline: 35.7089x; 1.0008x over previous
import jax
import jax.numpy as jnp
from jax import lax
from jax.experimental import pallas as pl

N_ANC = 2304
N_PAIR = 8
IB = 128
NMS_TH = 0.5


def _dot_relu_body(a_ref, b_ref, bias_ref, o_ref):
    acc = jnp.dot(a_ref[...], b_ref[...], preferred_element_type=jnp.float32)
    o_ref[...] = jnp.maximum(acc + bias_ref[...], 0.0)


def _dot_relu(a, b, bias, bm):
    m, k = a.shape
    n = b.shape[1]
    return pl.pallas_call(
        _dot_relu_body,
        grid=(m // bm,),
        in_specs=[pl.BlockSpec((bm, k), lambda i: (i, 0)),
                  pl.BlockSpec((k, n), lambda i: (0, 0)),
                  pl.BlockSpec((1, n), lambda i: (0, 0))],
        out_specs=pl.BlockSpec((bm, n), lambda i: (i, 0)),
        out_shape=jax.ShapeDtypeStruct((m, n), jnp.float32),
    )(a, b, bias)


def _tdot_relu_body(a_ref, b_ref, bias_ref, o_ref):
    acc = lax.dot_general(a_ref[...], b_ref[...], (((0,), (0,)), ((), ())),
                          preferred_element_type=jnp.float32)
    o_ref[...] = jnp.maximum(acc + bias_ref[...], 0.0)


def _tdot_relu(a_km, b_kn, bias, bm):
    k, m = a_km.shape
    n = b_kn.shape[1]
    return pl.pallas_call(
        _tdot_relu_body,
        grid=(m // bm,),
        in_specs=[pl.BlockSpec((k, bm), lambda i: (0, i)),
                  pl.BlockSpec((k, n), lambda i: (0, 0)),
                  pl.BlockSpec((1, n), lambda i: (0, 0))],
        out_specs=pl.BlockSpec((bm, n), lambda i: (i, 0)),
        out_shape=jax.ShapeDtypeStruct((m, n), jnp.float32),
    )(a_km, b_kn, bias)


def _tdot_relu_t_body(a_ref, b_ref, bias_ref, o_ref):
    acc = lax.dot_general(b_ref[...], a_ref[...], (((0,), (0,)), ((), ())),
                          preferred_element_type=jnp.float32)
    o_ref[...] = jnp.maximum(acc + bias_ref[...], 0.0)


def _tdot_relu_t(a_km, b_kn, bias_n1, bm):
    k, m = a_km.shape
    n = b_kn.shape[1]
    return pl.pallas_call(
        _tdot_relu_t_body,
        grid=(m // bm,),
        in_specs=[pl.BlockSpec((k, bm), lambda i: (0, i)),
                  pl.BlockSpec((k, n), lambda i: (0, 0)),
                  pl.BlockSpec((n, 1), lambda i: (0, 0))],
        out_specs=pl.BlockSpec((n, bm), lambda i: (0, i)),
        out_shape=jax.ShapeDtypeStruct((n, m), jnp.float32),
    )(a_km, b_kn, bias_n1)


def _patches_cm(x, leading_cb):
    d0, d1, h, w = x.shape
    c = d0 if leading_cb else d1
    ho, wo = h // 2, w // 2
    xp = jnp.pad(x, ((0, 0), (0, 0), (1, 1), (1, 1)))
    ev = xp[:, :, :, 0::2]
    od = xp[:, :, :, 1::2]
    wtap = {0: ev[..., 0:wo], 1: od[..., 0:wo], 2: ev[..., 1:wo + 1]}
    pieces = []
    for dy in range(3):
        for dx in range(3):
            pieces.append(wtap[dx][:, :, dy:dy + 2 * ho:2, :])
    g = jnp.stack(pieces, axis=0)
    if not leading_cb:
        g = g.transpose(0, 2, 1, 3, 4)
    return g.reshape(9 * c, -1)


def _wmat(w):
    o = w.shape[0]
    return w.transpose(2, 3, 1, 0).reshape(-1, o)


def _heads_body(f_ref, wc_ref, bc_ref, wr_ref, br_ref, g_ref, s_ref, bx_ref):
    f = f_ref[...]
    logits = jnp.dot(f, wc_ref[...], preferred_element_type=jnp.float32) + bc_ref[...]
    m = jnp.max(logits, axis=1, keepdims=True)
    e = jnp.exp(logits - m)
    s = (((e[:, 0:1] + e[:, 1:2]) + e[:, 2:3]) + e[:, 3:4]) + e[:, 4:5]
    s_ref[...] = e / s
    reg = jnp.dot(f, wr_ref[...], preferred_element_type=jnp.float32) + br_ref[...]
    bx_ref[...] = g_ref[...] + reg


def _heads(f3, wc, bc, wr, br, grid_boxes, bm=576):
    m = f3.shape[0]
    return pl.pallas_call(
        _heads_body,
        grid=(m // bm,),
        in_specs=[pl.BlockSpec((bm, 64), lambda i: (i, 0)),
                  pl.BlockSpec((64, 5), lambda i: (0, 0)),
                  pl.BlockSpec((1, 5), lambda i: (0, 0)),
                  pl.BlockSpec((64, 4), lambda i: (0, 0)),
                  pl.BlockSpec((1, 4), lambda i: (0, 0)),
                  pl.BlockSpec((bm, 4), lambda i: (i, 0))],
        out_specs=[pl.BlockSpec((bm, 5), lambda i: (i, 0)),
                   pl.BlockSpec((bm, 4), lambda i: (i, 0))],
        out_shape=[jax.ShapeDtypeStruct((m, 5), jnp.float32),
                   jax.ShapeDtypeStruct((m, 4), jnp.float32)],
    )(f3, wc, bc, wr, br, grid_boxes)


def _rank_body(s_ref, sall_ref, r_ref):
    ib = pl.program_id(0)
    si = s_ref[...][:, :, None]
    sj = sall_ref[...][:, None, :]
    i_idx = ib * IB + lax.broadcasted_iota(jnp.int32, (1, IB, 1), 1)
    j_idx = lax.broadcasted_iota(jnp.int32, (1, 1, N_ANC), 2)
    gt = (sj > si).astype(jnp.float32)
    tie = jnp.logical_and(sj == si, j_idx < i_idx).astype(jnp.float32)
    r_ref[...] = jnp.sum(gt + tie, axis=2)


def _ranks(scores_p):
    return pl.pallas_call(
        _rank_body,
        grid=(N_ANC // IB,),
        in_specs=[pl.BlockSpec((N_PAIR, IB), lambda i: (0, i)),
                  pl.BlockSpec((N_PAIR, N_ANC), lambda i: (0, 0))],
        out_specs=pl.BlockSpec((N_PAIR, IB), lambda i: (0, i)),
        out_shape=jax.ShapeDtypeStruct((N_PAIR, N_ANC), jnp.float32),
    )(scores_p, scores_p)


BIG = 1e9


SB = 8


def _nms_body(bxl_ref, bxt_ref, rank_ref, rankt_ref, keep_ref, keep2t_ref, keept_ref):
    keept_ref[...] = jnp.ones((N_ANC, N_PAIR), jnp.float32)
    rank_lane = rank_ref[...]

    def sweep_block(ib, t_lane):
        bt = bxt_ref[pl.ds(ib * SB, SB), :]
        rt = rankt_ref[pl.ds(ib * SB, SB), :]
        for img in range(2):
            c0 = img * 4
            x0i = bt[:, c0:c0 + 1]
            y0i = bt[:, c0 + 1:c0 + 2]
            x1i = bt[:, c0 + 2:c0 + 3]
            y1i = bt[:, c0 + 3:c0 + 4]
            ai = (x1i - x0i) * (y1i - y0i)
            x0j = bxl_ref[c0:c0 + 1, :]
            y0j = bxl_ref[c0 + 1:c0 + 2, :]
            x1j = bxl_ref[c0 + 2:c0 + 3, :]
            y1j = bxl_ref[c0 + 3:c0 + 4, :]
            aj = (x1j - x0j) * (y1j - y0j)
            iw = jnp.maximum(jnp.minimum(x1i, x1j) - jnp.maximum(x0i, x0j), 0.0)
            ih = jnp.maximum(jnp.minimum(y1i, y1j) - jnp.maximum(y0i, y0j), 0.0)
            inter = iw * ih
            u = ai + aj - inter + 1e-9
            cond = jnp.logical_and(inter + inter > u, u > 0.0)
            for q in range(4):
                p = img * 4 + q
                c = jnp.where(cond, t_lane[p:p + 1, :], BIG)
                m = jnp.min(c, axis=1, keepdims=True)
                newk = (m >= rt[:, p:p + 1]).astype(jnp.float32)
                keep2t_ref[pl.ds(ib * SB, SB), p:p + 1] = newk
        return t_lane

    def cond_fn(carry):
        it, changed = carry
        return jnp.logical_and(changed, it < N_ANC)

    def body_fn(carry):
        it, _ = carry
        kt = keept_ref[...]
        t_lane = jnp.where(jnp.transpose(kt) > 0.5, rank_lane, BIG)
        lax.fori_loop(0, N_ANC // SB, sweep_block, t_lane)
        diff = jnp.max(jnp.abs(keep2t_ref[...] - keept_ref[...]))
        keept_ref[...] = keep2t_ref[...]
        return it + 1, diff > 0.0

    lax.while_loop(cond_fn, body_fn, (jnp.int32(0), jnp.bool_(True)))
    keep_ref[...] = jnp.transpose(keept_ref[...])


def _nms_keep(boxes_l, boxes_t, ranks, ranks_t):
    from jax.experimental.pallas import tpu as pltpu
    keep, _ = pl.pallas_call(
        _nms_body,
        in_specs=[pl.BlockSpec((N_PAIR, N_ANC), lambda: (0, 0)),
                  pl.BlockSpec((N_ANC, 8), lambda: (0, 0)),
                  pl.BlockSpec((N_PAIR, N_ANC), lambda: (0, 0)),
                  pl.BlockSpec((N_ANC, N_PAIR), lambda: (0, 0))],
        out_specs=[pl.BlockSpec((N_PAIR, N_ANC), lambda: (0, 0)),
                   pl.BlockSpec((N_ANC, N_PAIR), lambda: (0, 0))],
        out_shape=[jax.ShapeDtypeStruct((N_PAIR, N_ANC), jnp.float32),
                   jax.ShapeDtypeStruct((N_ANC, N_PAIR), jnp.float32)],
        scratch_shapes=[pltpu.VMEM((N_ANC, N_PAIR), jnp.float32)],
    )(boxes_l, boxes_t, ranks, ranks_t)
    return keep


def _gather_body(planes_ref, rank_ref, o_ref):
    rb = pl.program_id(1)
    r_iota = (rb * IB + lax.broadcasted_iota(jnp.int32, (IB, 1), 0)).astype(jnp.float32)
    onehot = (rank_ref[0, 0, :][None, :] == r_iota).astype(jnp.bfloat16)
    x = planes_ref[0]
    hi = x.astype(jnp.bfloat16)
    r1 = x - hi.astype(jnp.float32)
    mid = r1.astype(jnp.bfloat16)
    lo = (r1 - mid.astype(jnp.float32)).astype(jnp.bfloat16)
    dn = (((1,), (1,)), ((), ()))
    gh = lax.dot_general(onehot, hi, dn, preferred_element_type=jnp.float32)
    gm = lax.dot_general(onehot, mid, dn, preferred_element_type=jnp.float32)
    gl = lax.dot_general(onehot, lo, dn, preferred_element_type=jnp.float32)
    g = gh + (gm + gl)
    keep_s = g[:, 5:6]
    o_ref[0, :, 0:5] = g[:, 0:5] * keep_s
    o_ref[0, :, 5:6] = keep_s


def _gather(planes, ranks):
    return pl.pallas_call(
        _gather_body,
        grid=(N_PAIR, N_ANC // IB),
        in_specs=[pl.BlockSpec((1, 6, N_ANC), lambda p, r: (p, 0, 0)),
                  pl.BlockSpec((1, 1, N_ANC), lambda p, r: (p, 0, 0))],
        out_specs=pl.BlockSpec((1, IB, 6), lambda p, r: (p, r, 0)),
        out_shape=jax.ShapeDtypeStruct((N_PAIR, N_ANC, 6), jnp.float32),
    )(planes, ranks)


def kernel(images, W1, b1, W2, b2, W3, b3, Wc, bc, Wr, br):
    p1 = _patches_cm(images, leading_cb=False)
    f1t = _tdot_relu_t(p1, _wmat(W1), b1[:, None], bm=4096)
    p2 = _patches_cm(f1t.reshape(16, 2, 192, 192), True)
    f2t = _tdot_relu_t(p2, _wmat(W2), b2[:, None], bm=2048)
    p3 = _patches_cm(f2t.reshape(32, 2, 96, 96), True)
    f3 = _tdot_relu(p3, _wmat(W3), b3[None, :], bm=512)

    gy, gx = jnp.meshgrid(jnp.arange(48), jnp.arange(48), indexing='ij')
    grid1 = jnp.stack([gx, gy, gx + 1, gy + 1], axis=-1).reshape(-1, 4).astype(jnp.float32)
    grid_b = jnp.tile(grid1, (2, 1))

    scores, boxes = _heads(f3, Wc.reshape(5, 64).T, bc[None, :],
                           Wr.reshape(4, 64).T, br[None, :], grid_b)

    s_img = scores.reshape(2, N_ANC, 5)
    scores_p = jnp.concatenate([s_img[0, :, 1:].T, s_img[1, :, 1:].T], axis=0)
    bx_img = boxes.reshape(2, N_ANC, 4)
    box_planes = jnp.stack([
        jnp.repeat(bx_img[:, :, q], 4, axis=0) for q in range(4)], axis=0)

    ranks = _ranks(scores_p)
    keep = _nms_keep(bx_img.transpose(0, 2, 1).reshape(8, N_ANC),
                     bx_img.transpose(1, 0, 2).reshape(N_ANC, 8),
                     ranks, ranks.T)

    planes = jnp.concatenate([box_planes,
                              scores_p[None, :, :],
                              keep[None, :, :]], axis=0)
    planes = planes.transpose(1, 0, 2)
    sorted_pl = _gather(planes, ranks.reshape(N_PAIR, 1, N_ANC))

    out_boxes = sorted_pl[:, :, 0:4].reshape(2, 4, N_ANC, 4)
    out_scores = sorted_pl[:, :, 4].reshape(2, 4, N_ANC)
    keep_mask = sorted_pl[:, :, 5].reshape(2, 4, N_ANC) > 0.5
    return out_boxes, out_scores, keep_mask
